# baseline (speedup 1.0000x reference)
import functools

import jax
import jax.numpy as jnp
from jax import lax
from jax.experimental import pallas as pl
from jax.experimental.pallas import tpu as pltpu
from jax.experimental.pallas import tpu_sc as plsc


def _make_sc_kernel(B, L, E, NC, NS):
    NW = NC * NS
    assert B % NW == 0
    RW = B // NW
    C = 4
    assert RW % C == 0
    NCHUNK = RW // C
    CL = C * L
    GS = 40
    assert L % GS == 0 and GS % 8 == 0
    NG = L // GS

    mesh = plsc.VectorSubcoreMesh(core_axis_name="c", subcore_axis_name="s")

    @functools.partial(
        pl.kernel,
        out_type=jax.ShapeDtypeStruct((B, L, E), jnp.float32),
        mesh=mesh,
        scratch_types=[
            pltpu.VMEM((L, E), jnp.float32),
            pltpu.VMEM((RW, L), jnp.int32),
            pltpu.VMEM((2, CL, E), jnp.float32),
            pltpu.SemaphoreType.DMA((2,)),
            pltpu.SemaphoreType.DMA((2,)),
        ],
        compiler_params=pltpu.CompilerParams(use_tc_tiling_on_sc=False),
    )
    def emb(tok_hbm, idx_hbm, pos_hbm, out_hbm, pos_v, idx_v, rows_v, gsem, osem):
        wid = lax.axis_index("s") * NC + lax.axis_index("c")
        b_base = wid * RW
        pltpu.sync_copy(idx_hbm.at[pl.ds(b_base, RW), :], idx_v)
        pltpu.sync_copy(pos_hbm, pos_v)

        def fire_gathers(g, s):
            return [
                pltpu.async_copy(
                    tok_hbm.at[idx_v.at[g * C + c, pl.ds(k * GS, GS)]],
                    rows_v.at[s, pl.ds(c * L + k * GS, GS), :],
                    gsem.at[s],
                )
                for c in range(C)
                for k in range(NG)
            ]

        def fire_out(g, s):
            return [
                pltpu.async_copy(
                    rows_v.at[s, pl.ds(c * L, L), :],
                    out_hbm.at[b_base + g * C + c],
                    osem.at[s],
                )
                for c in range(C)
            ]

        gather_descs = {0: fire_gathers(0, 0)}
        out_descs = {}
        for g in range(NCHUNK):
            s = g % 2
            if g + 1 < NCHUNK:
                if g >= 1:
                    for d in out_descs.pop(g - 1):
                        d.wait()
                gather_descs[g + 1] = fire_gathers(g + 1, 1 - s)
            for d in gather_descs.pop(g):
                d.wait()

            @pl.loop(0, L)
            def _row(j):
                p0 = pos_v[j, pl.ds(0, 16)]
                p1 = pos_v[j, pl.ds(16, 16)]
                for c in range(C):
                    r = c * L + j
                    rows_v[s, r, pl.ds(0, 16)] += p0
                    rows_v[s, r, pl.ds(16, 16)] += p1

            out_descs[g] = fire_out(g, s)
        for g in (NCHUNK - 2, NCHUNK - 1):
            for d in out_descs.pop(g):
                d.wait()

    return emb


def kernel(inputs, tok_table, pos_table):
    B, L = inputs.shape
    E = tok_table.shape[1]
    info = plsc.get_sparse_core_info()
    emb = _make_sc_kernel(B, L, E, info.num_cores, info.num_subcores)
    return emb(tok_table, inputs.astype(jnp.int32), pos_table)
